# baseline (speedup 1.0000x reference)
import functools

import jax
import jax.numpy as jnp
from jax import lax
from jax.experimental import pallas as pl
from jax.experimental.pallas import tpu as pltpu
from jax.experimental.pallas import tpu_sc as plsc

N = 10000
E = 320000
D = 128
H = 8
C = 16
G = 16
NEG_SLOPE = 0.2
BLK = 2000
GRID = N // BLK

DEN0 = N
NROW = N // 16
N2 = 10752


def _onehot(batch_blk):
    iota = lax.broadcasted_iota(jnp.int32, (BLK, G), 1)
    return (batch_blk == iota).astype(jnp.float32)


def _tc0_body(x_ref, w1_ref, b1_ref, w2_ref, b2_ref, wl_ref, bl_ref,
              wr_ref, br_ref, batch_ref, xl_ref, xr_ref, cnt_ref):
    i = pl.program_id(0)
    x = x_ref[...]
    h = jnp.maximum(jnp.dot(x, w1_ref[...], preferred_element_type=jnp.float32)
                    + b1_ref[...], 0.0)
    h = jnp.maximum(jnp.dot(h, w2_ref[...], preferred_element_type=jnp.float32)
                    + b2_ref[...], 0.0)
    xl_ref[...] = jnp.dot(h, wl_ref[...], preferred_element_type=jnp.float32) + bl_ref[...]
    xr_ref[...] = jnp.dot(h, wr_ref[...], preferred_element_type=jnp.float32) + br_ref[...]
    oh = _onehot(batch_ref[...])
    contrib = lax.dot_general(oh, jnp.ones((BLK, D), jnp.float32),
                              (((0,), (0,)), ((), ())),
                              preferred_element_type=jnp.float32)

    @pl.when(i == 0)
    def _():
        cnt_ref[...] = jnp.zeros_like(cnt_ref)

    cnt_ref[...] += contrib


def _tc0(x, w1, b1, w2, b2, wl, bl, wr, br, batch2d):
    full = lambda s: pl.BlockSpec(s, lambda i: tuple(0 for _ in s))
    return pl.pallas_call(
        _tc0_body,
        grid=(GRID,),
        in_specs=[
            pl.BlockSpec((BLK, D), lambda i: (i, 0)),
            full((D, D)), full((1, D)), full((D, D)), full((1, D)),
            full((D, D)), full((1, D)), full((D, D)), full((1, D)),
            pl.BlockSpec((BLK, 1), lambda i: (i, 0)),
        ],
        out_specs=[
            pl.BlockSpec((BLK, D), lambda i: (i, 0)),
            pl.BlockSpec((BLK, D), lambda i: (i, 0)),
            pl.BlockSpec((G, D), lambda i: (0, 0)),
        ],
        out_shape=[
            jax.ShapeDtypeStruct((N, D), jnp.float32),
            jax.ShapeDtypeStruct((N, D), jnp.float32),
            jax.ShapeDtypeStruct((G, D), jnp.float32),
        ],
    )(x, w1, b1, w2, b2, wl, bl, wr, br, batch2d)


def _norm_h(accn, den8, bo):
    numer = accn[0] + accn[1]
    den = den8[0] + den8[1]
    den_full = jnp.broadcast_to(den.reshape(BLK, H, 1), (BLK, H, C)).reshape(BLK, D)
    return jnp.maximum(numer / (den_full + 1e-16) + bo, 0.0)


def _tc_layer_body(accn_ref, den_ref, bo_ref, wl_ref, bl_ref, wr_ref, br_ref,
                   batch_ref, xl_ref, xr_ref, pool_ref):
    i = pl.program_id(0)
    h = _norm_h(accn_ref[...], den_ref[...], bo_ref[...])
    xl_ref[...] = jnp.dot(h, wl_ref[...], preferred_element_type=jnp.float32) + bl_ref[...]
    xr_ref[...] = jnp.dot(h, wr_ref[...], preferred_element_type=jnp.float32) + br_ref[...]
    oh = _onehot(batch_ref[...])
    contrib = lax.dot_general(oh, h, (((0,), (0,)), ((), ())),
                              preferred_element_type=jnp.float32)

    @pl.when(i == 0)
    def _():
        pool_ref[...] = jnp.zeros_like(pool_ref)

    pool_ref[...] += contrib


def _tc_layer(accn, den8, bo, wl, bl, wr, br, batch2d):
    full = lambda s: pl.BlockSpec(s, lambda i: tuple(0 for _ in s))
    return pl.pallas_call(
        _tc_layer_body,
        grid=(GRID,),
        in_specs=[
            pl.BlockSpec((2, BLK, D), lambda i: (0, i, 0)),
            pl.BlockSpec((2, BLK, H), lambda i: (0, i, 0)),
            full((1, D)),
            full((D, D)), full((1, D)), full((D, D)), full((1, D)),
            pl.BlockSpec((BLK, 1), lambda i: (i, 0)),
        ],
        out_specs=[
            pl.BlockSpec((BLK, D), lambda i: (i, 0)),
            pl.BlockSpec((BLK, D), lambda i: (i, 0)),
            pl.BlockSpec((G, D), lambda i: (0, 0)),
        ],
        out_shape=[
            jax.ShapeDtypeStruct((N, D), jnp.float32),
            jax.ShapeDtypeStruct((N, D), jnp.float32),
            jax.ShapeDtypeStruct((G, D), jnp.float32),
        ],
    )(accn, den8, bo, wl, bl, wr, br, batch2d)


def _tc_final_body(accn_ref, den_ref, bo_ref, batch_ref, p1_ref, p2_ref,
                   cnt_ref, out_ref, pool_ref):
    i = pl.program_id(0)
    h = _norm_h(accn_ref[...], den_ref[...], bo_ref[...])
    oh = _onehot(batch_ref[...])
    contrib = lax.dot_general(oh, h, (((0,), (0,)), ((), ())),
                              preferred_element_type=jnp.float32)

    @pl.when(i == 0)
    def _():
        pool_ref[...] = jnp.zeros_like(pool_ref)

    pool_ref[...] += contrib

    @pl.when(i == GRID - 1)
    def _():
        cnt = jnp.maximum(cnt_ref[...], 1.0)
        out_ref[...] = jnp.concatenate(
            [p1_ref[...] / cnt, p2_ref[...] / cnt, pool_ref[...] / cnt], axis=1)


def _tc_final(accn, den8, bo, batch2d, p1, p2, cnt):
    full = lambda s: pl.BlockSpec(s, lambda i: tuple(0 for _ in s))
    return pl.pallas_call(
        _tc_final_body,
        grid=(GRID,),
        in_specs=[
            pl.BlockSpec((2, BLK, D), lambda i: (0, i, 0)),
            pl.BlockSpec((2, BLK, H), lambda i: (0, i, 0)),
            full((1, D)),
            pl.BlockSpec((BLK, 1), lambda i: (i, 0)),
            full((G, D)), full((G, D)), full((G, D)),
        ],
        out_specs=[
            pl.BlockSpec((G, 3 * D), lambda i: (0, 0)),
            pl.BlockSpec((G, D), lambda i: (0, 0)),
        ],
        out_shape=[
            jax.ShapeDtypeStruct((G, 3 * D), jnp.float32),
            jax.ShapeDtypeStruct((G, D), jnp.float32),
        ],
    )(accn, den8, bo, batch2d, p1, p2, cnt)[0]


NC = 2
NS = 16
NT = NC * NS
EPT = E // NT
K = 80
NG = K // 16
SB = 400
CPS = SB // K
NSUPER = EPT // SB
TPT = N2 // NS
RB = 56


def _sc_edge_body(xl_h, xr_h, src_h, dst_h, ea_h, we_h, att_h, out_h,
                  srcb, dstb, eab, idx2, xs, xd, contrib2,
                  we_v, att_v, shared, sem1, sem2, sem3):
    cid = lax.axis_index("c")
    sid = lax.axis_index("s")
    wid = cid * NS + sid
    zeros16 = jnp.zeros((16,), jnp.float32)
    iota16 = lax.broadcasted_iota(jnp.int32, (16,), 0)
    rows = [iota16 + g * 16 for g in range(NG)]
    rows2 = [r * 2 for r in rows]

    @pl.loop(0, RB)
    def _(i):
        for j in range(D // 16):
            contrib2[i, pl.ds(j * 16, 16)] = zeros16

    @pl.loop(0, TPT // RB)
    def _(j):
        pltpu.sync_copy(contrib2.at[pl.ds(0, RB)],
                        shared.at[pl.ds(sid * TPT + j * RB, RB)])

    pltpu.sync_copy(we_h, we_v)
    pltpu.sync_copy(att_h, att_v)
    plsc.subcore_barrier()

    ebase = wid * EPT

    @pl.loop(0, NSUPER)
    def _(sc):
        sbase = ebase + sc * SB
        pltpu.sync_copy(src_h.at[pl.ds(sbase, SB)], srcb)
        pltpu.sync_copy(dst_h.at[pl.ds(sbase, SB)], dstb)
        pltpu.sync_copy(ea_h.at[pl.ds(sbase, SB)], eab)

        @pl.loop(0, CPS)
        def _(cc):
            co = cc * K
            d1 = pltpu.async_copy(xl_h.at[srcb.at[pl.ds(co, K)]], xs, sem1)
            d2 = pltpu.async_copy(xr_h.at[dstb.at[pl.ds(co, K)]], xd, sem2)

            @pl.when(jnp.logical_or(sc > 0, cc > 0))
            def _():
                pltpu.make_async_copy(contrib2, shared.at[idx2], sem3).wait()

            d1.wait()
            d2.wait()

            @pl.loop(0, K, unroll=8)
            def _(i):
                for j in range(D // 16):
                    contrib2[2 * i + 1, pl.ds(j * 16, 16)] = zeros16

            a_vecs = [eab[pl.ds(co + g * 16, 16)] for g in range(NG)]
            dvs = [dstb[pl.ds(co + g * 16, 16)] for g in range(NG)]
            dencols = [lax.shift_left(lax.bitwise_and(dv, 15), 3) for dv in dvs]
            for g in range(NG):
                plsc.store_scatter(idx2, [rows2[g]], dvs[g])
                plsc.store_scatter(idx2, [rows2[g] + 1],
                                   DEN0 + lax.shift_right_logical(dvs[g], 4))

            for h in range(H):
                col0 = h * C

                accs0 = tuple(jnp.zeros((16,), jnp.float32)
                              for _ in range(NG))

                @pl.loop(0, C, init_carry=accs0, unroll=8)
                def alpha_loop(c, accs, _col0=col0):
                    colv = jnp.full((16,), _col0 + c, jnp.int32)
                    web = plsc.load_gather(we_v, [colv])
                    atb = plsc.load_gather(att_v, [colv])
                    out = []
                    for g in range(NG):
                        xsc = plsc.load_gather(xs, [rows[g], colv])
                        xdc = plsc.load_gather(xd, [rows[g], colv])
                        e = xsc + xdc + a_vecs[g] * web
                        el = jnp.maximum(e, NEG_SLOPE * e)
                        out.append(accs[g] + el * atb)
                    return tuple(out)

                accs = alpha_loop
                exs = [jnp.exp(a) for a in accs]
                for g in range(NG):
                    plsc.store_scatter(contrib2, [rows2[g] + 1, dencols[g] + h],
                                       exs[g])

                @pl.loop(0, C, unroll=8)
                def _(c, _col0=col0, _exs=exs):
                    colv = jnp.full((16,), _col0 + c, jnp.int32)
                    for g in range(NG):
                        xsc = plsc.load_gather(xs, [rows[g], colv])
                        plsc.store_scatter(contrib2, [rows2[g], colv],
                                           xsc * _exs[g])

            pltpu.async_copy(contrib2, shared.at[idx2], sem3, add=True)

    pltpu.make_async_copy(contrib2, shared.at[idx2], sem3).wait()
    plsc.subcore_barrier()

    @pl.loop(0, TPT // RB)
    def _(j):
        r0 = sid * TPT + j * RB
        pltpu.sync_copy(shared.at[pl.ds(r0, RB)], contrib2.at[pl.ds(0, RB)])
        pltpu.sync_copy(contrib2.at[pl.ds(0, RB)], out_h.at[cid, pl.ds(r0, RB)])


def _edge_phase(xl, xr, src, dst, ea, we_flat, att_flat):
    mesh = plsc.VectorSubcoreMesh(core_axis_name="c", subcore_axis_name="s")
    f = pl.kernel(
        _sc_edge_body,
        out_type=jax.ShapeDtypeStruct((NC, N2, D), jnp.float32),
        mesh=mesh,
        compiler_params=pltpu.CompilerParams(needs_layout_passes=False),
        scratch_types=[
            pltpu.VMEM((SB,), jnp.int32),
            pltpu.VMEM((SB,), jnp.int32),
            pltpu.VMEM((SB,), jnp.float32),
            pltpu.VMEM((2 * K,), jnp.int32),
            pltpu.VMEM((K, D), jnp.float32),
            pltpu.VMEM((K, D), jnp.float32),
            pltpu.VMEM((2 * K, D), jnp.float32),
            pltpu.VMEM((D,), jnp.float32),
            pltpu.VMEM((D,), jnp.float32),
            pltpu.VMEM_SHARED((N2, D), jnp.float32),
            pltpu.SemaphoreType.DMA,
            pltpu.SemaphoreType.DMA,
            pltpu.SemaphoreType.DMA,
        ],
    )
    return f(xl, xr, src, dst, ea, we_flat.reshape(D), att_flat.reshape(D))


def kernel(x, edge_index, edge_attr, batch, W_pre1, b_pre1, W_pre2, b_pre2,
           Wl0, bl0, Wr0, br0, We0, att0, bo0,
           Wl1, bl1, Wr1, br1, We1, att1, bo1,
           Wl2, bl2, Wr2, br2, We2, att2, bo2):
    src = edge_index[0]
    dst = edge_index[1]
    ea = edge_attr.reshape(E)
    batch2d = batch.reshape(N, 1)
    r = lambda b: b.reshape(1, D)

    def split_acc(acc):
        accn = acc[:, :N, :]
        den8 = acc[:, DEN0:DEN0 + NROW, :].reshape(2, NROW, 16, H).reshape(2, N, H)
        return accn, den8

    xl, xr, cnt = _tc0(x, W_pre1, r(b_pre1), W_pre2, r(b_pre2),
                       Wl0, r(bl0), Wr0, r(br0), batch2d)

    accn, den8 = split_acc(_edge_phase(xl, xr, src, dst, ea, We0, att0))
    xl, xr, p1 = _tc_layer(accn, den8, r(bo0), Wl1, r(bl1), Wr1, r(br1), batch2d)

    accn, den8 = split_acc(_edge_phase(xl, xr, src, dst, ea, We1, att1))
    xl, xr, p2 = _tc_layer(accn, den8, r(bo1), Wl2, r(bl2), Wr2, r(br2), batch2d)

    accn, den8 = split_acc(_edge_phase(xl, xr, src, dst, ea, We2, att2))
    return _tc_final(accn, den8, r(bo2), batch2d, p1, p2, cnt)


# speedup vs baseline: 3.0077x; 3.0077x over previous
import functools

import jax
import jax.numpy as jnp
from jax import lax
from jax.experimental import pallas as pl
from jax.experimental.pallas import tpu as pltpu
from jax.experimental.pallas import tpu_sc as plsc

N = 10000
E = 320000
D = 128
H = 8
C = 16
G = 16
NEG_SLOPE = 0.2
BLK = 2000
GRID = N // BLK

DEN0 = N
NROW = N // 16
N2 = 10752


def _onehot(batch_blk):
    iota = lax.broadcasted_iota(jnp.int32, (BLK, G), 1)
    return (batch_blk == iota).astype(jnp.float32)


def _tc0_body(x_ref, w1_ref, b1_ref, w2_ref, b2_ref, wl_ref, bl_ref,
              wr_ref, br_ref, batch_ref, xl_ref, xr_ref, cnt_ref):
    i = pl.program_id(0)
    x = x_ref[...]
    h = jnp.maximum(jnp.dot(x, w1_ref[...], preferred_element_type=jnp.float32)
                    + b1_ref[...], 0.0)
    h = jnp.maximum(jnp.dot(h, w2_ref[...], preferred_element_type=jnp.float32)
                    + b2_ref[...], 0.0)
    xl_ref[...] = jnp.dot(h, wl_ref[...], preferred_element_type=jnp.float32) + bl_ref[...]
    xr_ref[...] = jnp.dot(h, wr_ref[...], preferred_element_type=jnp.float32) + br_ref[...]
    oh = _onehot(batch_ref[...])
    contrib = lax.dot_general(oh, jnp.ones((BLK, D), jnp.float32),
                              (((0,), (0,)), ((), ())),
                              preferred_element_type=jnp.float32)

    @pl.when(i == 0)
    def _():
        cnt_ref[...] = jnp.zeros_like(cnt_ref)

    cnt_ref[...] += contrib


def _tc0(x, w1, b1, w2, b2, wl, bl, wr, br, batch2d):
    full = lambda s: pl.BlockSpec(s, lambda i: tuple(0 for _ in s))
    return pl.pallas_call(
        _tc0_body,
        grid=(GRID,),
        in_specs=[
            pl.BlockSpec((BLK, D), lambda i: (i, 0)),
            full((D, D)), full((1, D)), full((D, D)), full((1, D)),
            full((D, D)), full((1, D)), full((D, D)), full((1, D)),
            pl.BlockSpec((BLK, 1), lambda i: (i, 0)),
        ],
        out_specs=[
            pl.BlockSpec((BLK, D), lambda i: (i, 0)),
            pl.BlockSpec((BLK, D), lambda i: (i, 0)),
            pl.BlockSpec((G, D), lambda i: (0, 0)),
        ],
        out_shape=[
            jax.ShapeDtypeStruct((N, D), jnp.float32),
            jax.ShapeDtypeStruct((N, D), jnp.float32),
            jax.ShapeDtypeStruct((G, D), jnp.float32),
        ],
    )(x, w1, b1, w2, b2, wl, bl, wr, br, batch2d)


def _norm_h(accn, den8, bo):
    numer = accn[0] + accn[1]
    den = den8[0] + den8[1]
    den_full = jnp.broadcast_to(den.reshape(BLK, H, 1), (BLK, H, C)).reshape(BLK, D)
    return jnp.maximum(numer / (den_full + 1e-16) + bo, 0.0)


def _tc_layer_body(accn_ref, den_ref, bo_ref, wl_ref, bl_ref, wr_ref, br_ref,
                   batch_ref, xl_ref, xr_ref, pool_ref):
    i = pl.program_id(0)
    h = _norm_h(accn_ref[...], den_ref[...], bo_ref[...])
    xl_ref[...] = jnp.dot(h, wl_ref[...], preferred_element_type=jnp.float32) + bl_ref[...]
    xr_ref[...] = jnp.dot(h, wr_ref[...], preferred_element_type=jnp.float32) + br_ref[...]
    oh = _onehot(batch_ref[...])
    contrib = lax.dot_general(oh, h, (((0,), (0,)), ((), ())),
                              preferred_element_type=jnp.float32)

    @pl.when(i == 0)
    def _():
        pool_ref[...] = jnp.zeros_like(pool_ref)

    pool_ref[...] += contrib


def _tc_layer(accn, den8, bo, wl, bl, wr, br, batch2d):
    full = lambda s: pl.BlockSpec(s, lambda i: tuple(0 for _ in s))
    return pl.pallas_call(
        _tc_layer_body,
        grid=(GRID,),
        in_specs=[
            pl.BlockSpec((2, BLK, D), lambda i: (0, i, 0)),
            pl.BlockSpec((2, BLK, H), lambda i: (0, i, 0)),
            full((1, D)),
            full((D, D)), full((1, D)), full((D, D)), full((1, D)),
            pl.BlockSpec((BLK, 1), lambda i: (i, 0)),
        ],
        out_specs=[
            pl.BlockSpec((BLK, D), lambda i: (i, 0)),
            pl.BlockSpec((BLK, D), lambda i: (i, 0)),
            pl.BlockSpec((G, D), lambda i: (0, 0)),
        ],
        out_shape=[
            jax.ShapeDtypeStruct((N, D), jnp.float32),
            jax.ShapeDtypeStruct((N, D), jnp.float32),
            jax.ShapeDtypeStruct((G, D), jnp.float32),
        ],
    )(accn, den8, bo, wl, bl, wr, br, batch2d)


def _tc_final_body(accn_ref, den_ref, bo_ref, batch_ref, p1_ref, p2_ref,
                   cnt_ref, out_ref, pool_ref):
    i = pl.program_id(0)
    h = _norm_h(accn_ref[...], den_ref[...], bo_ref[...])
    oh = _onehot(batch_ref[...])
    contrib = lax.dot_general(oh, h, (((0,), (0,)), ((), ())),
                              preferred_element_type=jnp.float32)

    @pl.when(i == 0)
    def _():
        pool_ref[...] = jnp.zeros_like(pool_ref)

    pool_ref[...] += contrib

    @pl.when(i == GRID - 1)
    def _():
        cnt = jnp.maximum(cnt_ref[...], 1.0)
        out_ref[...] = jnp.concatenate(
            [p1_ref[...] / cnt, p2_ref[...] / cnt, pool_ref[...] / cnt], axis=1)


def _tc_final(accn, den8, bo, batch2d, p1, p2, cnt):
    full = lambda s: pl.BlockSpec(s, lambda i: tuple(0 for _ in s))
    return pl.pallas_call(
        _tc_final_body,
        grid=(GRID,),
        in_specs=[
            pl.BlockSpec((2, BLK, D), lambda i: (0, i, 0)),
            pl.BlockSpec((2, BLK, H), lambda i: (0, i, 0)),
            full((1, D)),
            pl.BlockSpec((BLK, 1), lambda i: (i, 0)),
            full((G, D)), full((G, D)), full((G, D)),
        ],
        out_specs=[
            pl.BlockSpec((G, 3 * D), lambda i: (0, 0)),
            pl.BlockSpec((G, D), lambda i: (0, 0)),
        ],
        out_shape=[
            jax.ShapeDtypeStruct((G, 3 * D), jnp.float32),
            jax.ShapeDtypeStruct((G, D), jnp.float32),
        ],
    )(accn, den8, bo, batch2d, p1, p2, cnt)[0]


NC = 2
NS = 16
NT = NC * NS
EPT = E // NT
K = 80
NG = K // 16
SB = 400
CPS = SB // K
NSUPER = EPT // SB
TPT = N2 // NS
RB = 56


def _sc_edge_body(xl_h, xr_h, src_h, dst_h, ea_h, we_h, att_h, out_h,
                  srcb, dstb, eab, idx2, xs, xd, contrib2,
                  we_v, att_v, shared, sem1, sem2, sem3):
    cid = lax.axis_index("c")
    sid = lax.axis_index("s")
    wid = cid * NS + sid
    zeros16 = jnp.zeros((16,), jnp.float32)
    iota16 = lax.broadcasted_iota(jnp.int32, (16,), 0)
    rows = [iota16 + g * 16 for g in range(NG)]
    rows2 = [r * 2 for r in rows]

    @pl.loop(0, RB)
    def _(i):
        for j in range(D // 16):
            contrib2[i, pl.ds(j * 16, 16)] = zeros16

    @pl.loop(0, TPT // RB)
    def _(j):
        pltpu.sync_copy(contrib2.at[pl.ds(0, RB)],
                        shared.at[pl.ds(sid * TPT + j * RB, RB)])

    pltpu.sync_copy(we_h, we_v)
    pltpu.sync_copy(att_h, att_v)
    plsc.subcore_barrier()

    ebase = wid * EPT

    @pl.loop(0, NSUPER)
    def _(sc):
        sbase = ebase + sc * SB
        pltpu.sync_copy(src_h.at[pl.ds(sbase, SB)], srcb)
        pltpu.sync_copy(dst_h.at[pl.ds(sbase, SB)], dstb)
        pltpu.sync_copy(ea_h.at[pl.ds(sbase, SB)], eab)

        @pl.loop(0, CPS)
        def _(cc):
            co = cc * K
            d1 = pltpu.async_copy(xl_h.at[srcb.at[pl.ds(co, K)]], xs, sem1)
            d2 = pltpu.async_copy(xr_h.at[dstb.at[pl.ds(co, K)]], xd, sem2)

            @pl.when(jnp.logical_or(sc > 0, cc > 0))
            def _():
                pltpu.make_async_copy(contrib2, shared.at[idx2], sem3).wait()

            d1.wait()
            d2.wait()

            @pl.loop(0, K, unroll=8)
            def _(i):
                for j in range(D // 16):
                    contrib2[2 * i + 1, pl.ds(j * 16, 16)] = zeros16

            a_vecs = [eab[pl.ds(co + g * 16, 16)] for g in range(NG)]
            dvs = [dstb[pl.ds(co + g * 16, 16)] for g in range(NG)]
            dencols = [lax.shift_left(lax.bitwise_and(dv, 15), 3) for dv in dvs]
            for g in range(NG):
                plsc.store_scatter(idx2, [rows2[g]], dvs[g])
                plsc.store_scatter(idx2, [rows2[g] + 1],
                                   DEN0 + lax.shift_right_logical(dvs[g], 4))

            for h in range(H):
                col0 = h * C

                accs0 = tuple(jnp.zeros((16,), jnp.float32)
                              for _ in range(NG))

                @pl.loop(0, C, init_carry=accs0, unroll=8)
                def alpha_loop(c, accs, _col0=col0):
                    colv = lax.bitwise_and(iota16 + c, C - 1) + _col0
                    web = plsc.load_gather(we_v, [colv])
                    atb = plsc.load_gather(att_v, [colv])
                    out = []
                    for g in range(NG):
                        xsc = plsc.load_gather(xs, [rows[g], colv])
                        xdc = plsc.load_gather(xd, [rows[g], colv])
                        e = xsc + xdc + a_vecs[g] * web
                        el = jnp.maximum(e, NEG_SLOPE * e)
                        out.append(accs[g] + el * atb)
                    return tuple(out)

                accs = alpha_loop
                exs = [jnp.exp(a) for a in accs]
                for g in range(NG):
                    plsc.store_scatter(contrib2, [rows2[g] + 1, dencols[g] + h],
                                       exs[g])

                @pl.loop(0, C, unroll=8)
                def _(c, _col0=col0, _exs=exs):
                    colv = lax.bitwise_and(iota16 + c, C - 1) + _col0
                    for g in range(NG):
                        xsc = plsc.load_gather(xs, [rows[g], colv])
                        plsc.store_scatter(contrib2, [rows2[g], colv],
                                           xsc * _exs[g])

            pltpu.async_copy(contrib2, shared.at[idx2], sem3, add=True)

    pltpu.make_async_copy(contrib2, shared.at[idx2], sem3).wait()
    plsc.subcore_barrier()

    @pl.loop(0, TPT // RB)
    def _(j):
        r0 = sid * TPT + j * RB
        pltpu.sync_copy(shared.at[pl.ds(r0, RB)], contrib2.at[pl.ds(0, RB)])
        pltpu.sync_copy(contrib2.at[pl.ds(0, RB)], out_h.at[cid, pl.ds(r0, RB)])


def _edge_phase(xl, xr, src, dst, ea, we_flat, att_flat):
    mesh = plsc.VectorSubcoreMesh(core_axis_name="c", subcore_axis_name="s")
    f = pl.kernel(
        _sc_edge_body,
        out_type=jax.ShapeDtypeStruct((NC, N2, D), jnp.float32),
        mesh=mesh,
        compiler_params=pltpu.CompilerParams(needs_layout_passes=False),
        scratch_types=[
            pltpu.VMEM((SB,), jnp.int32),
            pltpu.VMEM((SB,), jnp.int32),
            pltpu.VMEM((SB,), jnp.float32),
            pltpu.VMEM((2 * K,), jnp.int32),
            pltpu.VMEM((K, D), jnp.float32),
            pltpu.VMEM((K, D), jnp.float32),
            pltpu.VMEM((2 * K, D), jnp.float32),
            pltpu.VMEM((D,), jnp.float32),
            pltpu.VMEM((D,), jnp.float32),
            pltpu.VMEM_SHARED((N2, D), jnp.float32),
            pltpu.SemaphoreType.DMA,
            pltpu.SemaphoreType.DMA,
            pltpu.SemaphoreType.DMA,
        ],
    )
    return f(xl, xr, src, dst, ea, we_flat.reshape(D), att_flat.reshape(D))


def kernel(x, edge_index, edge_attr, batch, W_pre1, b_pre1, W_pre2, b_pre2,
           Wl0, bl0, Wr0, br0, We0, att0, bo0,
           Wl1, bl1, Wr1, br1, We1, att1, bo1,
           Wl2, bl2, Wr2, br2, We2, att2, bo2):
    src = edge_index[0]
    dst = edge_index[1]
    ea = edge_attr.reshape(E)
    batch2d = batch.reshape(N, 1)
    r = lambda b: b.reshape(1, D)

    def split_acc(acc):
        accn = acc[:, :N, :]
        den8 = acc[:, DEN0:DEN0 + NROW, :].reshape(2, NROW, 16, H).reshape(2, N, H)
        return accn, den8

    xl, xr, cnt = _tc0(x, W_pre1, r(b_pre1), W_pre2, r(b_pre2),
                       Wl0, r(bl0), Wr0, r(br0), batch2d)

    accn, den8 = split_acc(_edge_phase(xl, xr, src, dst, ea, We0, att0))
    xl, xr, p1 = _tc_layer(accn, den8, r(bo0), Wl1, r(bl1), Wr1, r(br1), batch2d)

    accn, den8 = split_acc(_edge_phase(xl, xr, src, dst, ea, We1, att1))
    xl, xr, p2 = _tc_layer(accn, den8, r(bo1), Wl2, r(bl2), Wr2, r(br2), batch2d)

    accn, den8 = split_acc(_edge_phase(xl, xr, src, dst, ea, We2, att2))
    return _tc_final(accn, den8, r(bo2), batch2d, p1, p2, cnt)


# speedup vs baseline: 4.5594x; 1.5159x over previous
import functools

import jax
import jax.numpy as jnp
from jax import lax
from jax.experimental import pallas as pl
from jax.experimental.pallas import tpu as pltpu
from jax.experimental.pallas import tpu_sc as plsc

N = 10000
E = 320000
D = 128
H = 8
C = 16
G = 16
NEG_SLOPE = 0.2
BLK = 2000
GRID = N // BLK

DEN0 = N
NROW = N // 16
N2 = 10752


def _onehot(batch_blk):
    iota = lax.broadcasted_iota(jnp.int32, (BLK, G), 1)
    return (batch_blk == iota).astype(jnp.float32)


def _tc0_body(x_ref, w1_ref, b1_ref, w2_ref, b2_ref, wl_ref, bl_ref,
              wr_ref, br_ref, batch_ref, xl_ref, xr_ref, cnt_ref):
    i = pl.program_id(0)
    x = x_ref[...]
    h = jnp.maximum(jnp.dot(x, w1_ref[...], preferred_element_type=jnp.float32)
                    + b1_ref[...], 0.0)
    h = jnp.maximum(jnp.dot(h, w2_ref[...], preferred_element_type=jnp.float32)
                    + b2_ref[...], 0.0)
    xl_ref[...] = jnp.dot(h, wl_ref[...], preferred_element_type=jnp.float32) + bl_ref[...]
    xr_ref[...] = jnp.dot(h, wr_ref[...], preferred_element_type=jnp.float32) + br_ref[...]
    oh = _onehot(batch_ref[...])
    contrib = lax.dot_general(oh, jnp.ones((BLK, D), jnp.float32),
                              (((0,), (0,)), ((), ())),
                              preferred_element_type=jnp.float32)

    @pl.when(i == 0)
    def _():
        cnt_ref[...] = jnp.zeros_like(cnt_ref)

    cnt_ref[...] += contrib


def _tc0(x, w1, b1, w2, b2, wl, bl, wr, br, batch2d):
    full = lambda s: pl.BlockSpec(s, lambda i: tuple(0 for _ in s))
    return pl.pallas_call(
        _tc0_body,
        grid=(GRID,),
        in_specs=[
            pl.BlockSpec((BLK, D), lambda i: (i, 0)),
            full((D, D)), full((1, D)), full((D, D)), full((1, D)),
            full((D, D)), full((1, D)), full((D, D)), full((1, D)),
            pl.BlockSpec((BLK, 1), lambda i: (i, 0)),
        ],
        out_specs=[
            pl.BlockSpec((BLK, D), lambda i: (i, 0)),
            pl.BlockSpec((BLK, D), lambda i: (i, 0)),
            pl.BlockSpec((G, D), lambda i: (0, 0)),
        ],
        out_shape=[
            jax.ShapeDtypeStruct((N, D), jnp.float32),
            jax.ShapeDtypeStruct((N, D), jnp.float32),
            jax.ShapeDtypeStruct((G, D), jnp.float32),
        ],
    )(x, w1, b1, w2, b2, wl, bl, wr, br, batch2d)


def _norm_h(accn, den8, bo):
    numer = accn[0] + accn[1]
    den = den8[0] + den8[1]
    den_full = jnp.broadcast_to(den.reshape(BLK, H, 1), (BLK, H, C)).reshape(BLK, D)
    return jnp.maximum(numer / (den_full + 1e-16) + bo, 0.0)


def _tc_layer_body(accn_ref, den_ref, bo_ref, wl_ref, bl_ref, wr_ref, br_ref,
                   batch_ref, xl_ref, xr_ref, pool_ref):
    i = pl.program_id(0)
    h = _norm_h(accn_ref[...], den_ref[...], bo_ref[...])
    xl_ref[...] = jnp.dot(h, wl_ref[...], preferred_element_type=jnp.float32) + bl_ref[...]
    xr_ref[...] = jnp.dot(h, wr_ref[...], preferred_element_type=jnp.float32) + br_ref[...]
    oh = _onehot(batch_ref[...])
    contrib = lax.dot_general(oh, h, (((0,), (0,)), ((), ())),
                              preferred_element_type=jnp.float32)

    @pl.when(i == 0)
    def _():
        pool_ref[...] = jnp.zeros_like(pool_ref)

    pool_ref[...] += contrib


def _tc_layer(accn, den8, bo, wl, bl, wr, br, batch2d):
    full = lambda s: pl.BlockSpec(s, lambda i: tuple(0 for _ in s))
    return pl.pallas_call(
        _tc_layer_body,
        grid=(GRID,),
        in_specs=[
            pl.BlockSpec((2, BLK, D), lambda i: (0, i, 0)),
            pl.BlockSpec((2, BLK, H), lambda i: (0, i, 0)),
            full((1, D)),
            full((D, D)), full((1, D)), full((D, D)), full((1, D)),
            pl.BlockSpec((BLK, 1), lambda i: (i, 0)),
        ],
        out_specs=[
            pl.BlockSpec((BLK, D), lambda i: (i, 0)),
            pl.BlockSpec((BLK, D), lambda i: (i, 0)),
            pl.BlockSpec((G, D), lambda i: (0, 0)),
        ],
        out_shape=[
            jax.ShapeDtypeStruct((N, D), jnp.float32),
            jax.ShapeDtypeStruct((N, D), jnp.float32),
            jax.ShapeDtypeStruct((G, D), jnp.float32),
        ],
    )(accn, den8, bo, wl, bl, wr, br, batch2d)


def _tc_final_body(accn_ref, den_ref, bo_ref, batch_ref, p1_ref, p2_ref,
                   cnt_ref, out_ref, pool_ref):
    i = pl.program_id(0)
    h = _norm_h(accn_ref[...], den_ref[...], bo_ref[...])
    oh = _onehot(batch_ref[...])
    contrib = lax.dot_general(oh, h, (((0,), (0,)), ((), ())),
                              preferred_element_type=jnp.float32)

    @pl.when(i == 0)
    def _():
        pool_ref[...] = jnp.zeros_like(pool_ref)

    pool_ref[...] += contrib

    @pl.when(i == GRID - 1)
    def _():
        cnt = jnp.maximum(cnt_ref[...], 1.0)
        out_ref[...] = jnp.concatenate(
            [p1_ref[...] / cnt, p2_ref[...] / cnt, pool_ref[...] / cnt], axis=1)


def _tc_final(accn, den8, bo, batch2d, p1, p2, cnt):
    full = lambda s: pl.BlockSpec(s, lambda i: tuple(0 for _ in s))
    return pl.pallas_call(
        _tc_final_body,
        grid=(GRID,),
        in_specs=[
            pl.BlockSpec((2, BLK, D), lambda i: (0, i, 0)),
            pl.BlockSpec((2, BLK, H), lambda i: (0, i, 0)),
            full((1, D)),
            pl.BlockSpec((BLK, 1), lambda i: (i, 0)),
            full((G, D)), full((G, D)), full((G, D)),
        ],
        out_specs=[
            pl.BlockSpec((G, 3 * D), lambda i: (0, 0)),
            pl.BlockSpec((G, D), lambda i: (0, 0)),
        ],
        out_shape=[
            jax.ShapeDtypeStruct((G, 3 * D), jnp.float32),
            jax.ShapeDtypeStruct((G, D), jnp.float32),
        ],
    )(accn, den8, bo, batch2d, p1, p2, cnt)[0]


NC = 2
NS = 16
NT = NC * NS
EPT = E // NT
K = 80
NG = K // 16
SB = 400
CPS = SB // K
NSUPER = EPT // SB
TPT = N2 // NS
RB = 56


def _sc_edge_body(xl_h, xr_h, src_h, dst_h, ea_h, we_h, att_h, out_h,
                  srcb, dstb, eab, idx2, xs, xd, contrib2,
                  we_v, att_v, shared, sem1, sem2, sem3):
    cid = lax.axis_index("c")
    sid = lax.axis_index("s")
    wid = cid * NS + sid
    zeros16 = jnp.zeros((16,), jnp.float32)
    iota16 = lax.broadcasted_iota(jnp.int32, (16,), 0)
    rows = [iota16 + g * 16 for g in range(NG)]
    rows2 = [r * 2 for r in rows]

    @pl.loop(0, RB)
    def _(i):
        for j in range(D // 16):
            contrib2[i, pl.ds(j * 16, 16)] = zeros16

    @pl.loop(0, TPT // RB)
    def _(j):
        pltpu.sync_copy(contrib2.at[pl.ds(0, RB)],
                        shared.at[pl.ds(sid * TPT + j * RB, RB)])

    pltpu.sync_copy(we_h, we_v)
    pltpu.sync_copy(att_h, att_v)
    plsc.subcore_barrier()

    ebase = wid * EPT

    @pl.loop(0, NSUPER)
    def _(sc):
        sbase = ebase + sc * SB
        pltpu.sync_copy(src_h.at[pl.ds(sbase, SB)], srcb)
        pltpu.sync_copy(dst_h.at[pl.ds(sbase, SB)], dstb)
        pltpu.sync_copy(ea_h.at[pl.ds(sbase, SB)], eab)

        @pl.loop(0, CPS)
        def _(cc):
            co = cc * K
            d1 = pltpu.async_copy(xl_h.at[srcb.at[pl.ds(co, K)]], xs, sem1)
            d2 = pltpu.async_copy(xr_h.at[dstb.at[pl.ds(co, K)]], xd, sem2)

            @pl.when(jnp.logical_or(sc > 0, cc > 0))
            def _():
                pltpu.make_async_copy(contrib2, shared.at[idx2], sem3).wait()

            d1.wait()
            d2.wait()

            @plsc.parallel_loop(0, K, unroll=8)
            def _(i):
                for j in range(D // 16):
                    contrib2[2 * i + 1, pl.ds(j * 16, 16)] = zeros16

            a_vecs = [eab[pl.ds(co + g * 16, 16)] for g in range(NG)]
            dvs = [dstb[pl.ds(co + g * 16, 16)] for g in range(NG)]
            dencols = [lax.shift_left(lax.bitwise_and(dv, 15), 3) for dv in dvs]
            for g in range(NG):
                plsc.store_scatter(idx2, [rows2[g]], dvs[g])
                plsc.store_scatter(idx2, [rows2[g] + 1],
                                   DEN0 + lax.shift_right_logical(dvs[g], 4))

            for h in range(H):
                col0 = h * C

                accs0 = tuple(jnp.zeros((16,), jnp.float32)
                              for _ in range(NG))

                @plsc.parallel_loop(0, C, unroll=8, carry=accs0)
                def alpha_loop(c, accs, _col0=col0):
                    colv = lax.bitwise_and(iota16 + c, C - 1) + _col0
                    web = plsc.load_gather(we_v, [colv])
                    atb = plsc.load_gather(att_v, [colv])
                    out = []
                    for g in range(NG):
                        xsc = plsc.load_gather(xs, [rows[g], colv])
                        xdc = plsc.load_gather(xd, [rows[g], colv])
                        e = xsc + xdc + a_vecs[g] * web
                        el = jnp.maximum(e, NEG_SLOPE * e)
                        out.append(accs[g] + el * atb)
                    return tuple(out)

                accs = alpha_loop
                exs = [jnp.exp(a) for a in accs]
                for g in range(NG):
                    plsc.store_scatter(contrib2, [rows2[g] + 1, dencols[g] + h],
                                       exs[g])

                @plsc.parallel_loop(0, C, unroll=8)
                def _(c, _col0=col0, _exs=exs):
                    colv = lax.bitwise_and(iota16 + c, C - 1) + _col0
                    for g in range(NG):
                        xsc = plsc.load_gather(xs, [rows[g], colv])
                        plsc.store_scatter(contrib2, [rows2[g], colv],
                                           xsc * _exs[g])

            pltpu.async_copy(contrib2, shared.at[idx2], sem3, add=True)

    pltpu.make_async_copy(contrib2, shared.at[idx2], sem3).wait()
    plsc.subcore_barrier()

    @pl.loop(0, TPT // RB)
    def _(j):
        r0 = sid * TPT + j * RB
        pltpu.sync_copy(shared.at[pl.ds(r0, RB)], contrib2.at[pl.ds(0, RB)])
        pltpu.sync_copy(contrib2.at[pl.ds(0, RB)], out_h.at[cid, pl.ds(r0, RB)])


def _edge_phase(xl, xr, src, dst, ea, we_flat, att_flat):
    mesh = plsc.VectorSubcoreMesh(core_axis_name="c", subcore_axis_name="s")
    f = pl.kernel(
        _sc_edge_body,
        out_type=jax.ShapeDtypeStruct((NC, N2, D), jnp.float32),
        mesh=mesh,
        compiler_params=pltpu.CompilerParams(needs_layout_passes=False),
        scratch_types=[
            pltpu.VMEM((SB,), jnp.int32),
            pltpu.VMEM((SB,), jnp.int32),
            pltpu.VMEM((SB,), jnp.float32),
            pltpu.VMEM((2 * K,), jnp.int32),
            pltpu.VMEM((K, D), jnp.float32),
            pltpu.VMEM((K, D), jnp.float32),
            pltpu.VMEM((2 * K, D), jnp.float32),
            pltpu.VMEM((D,), jnp.float32),
            pltpu.VMEM((D,), jnp.float32),
            pltpu.VMEM_SHARED((N2, D), jnp.float32),
            pltpu.SemaphoreType.DMA,
            pltpu.SemaphoreType.DMA,
            pltpu.SemaphoreType.DMA,
        ],
    )
    return f(xl, xr, src, dst, ea, we_flat.reshape(D), att_flat.reshape(D))


def kernel(x, edge_index, edge_attr, batch, W_pre1, b_pre1, W_pre2, b_pre2,
           Wl0, bl0, Wr0, br0, We0, att0, bo0,
           Wl1, bl1, Wr1, br1, We1, att1, bo1,
           Wl2, bl2, Wr2, br2, We2, att2, bo2):
    src = edge_index[0]
    dst = edge_index[1]
    ea = edge_attr.reshape(E)
    batch2d = batch.reshape(N, 1)
    r = lambda b: b.reshape(1, D)

    def split_acc(acc):
        accn = acc[:, :N, :]
        den8 = acc[:, DEN0:DEN0 + NROW, :].reshape(2, NROW, 16, H).reshape(2, N, H)
        return accn, den8

    xl, xr, cnt = _tc0(x, W_pre1, r(b_pre1), W_pre2, r(b_pre2),
                       Wl0, r(bl0), Wr0, r(br0), batch2d)

    accn, den8 = split_acc(_edge_phase(xl, xr, src, dst, ea, We0, att0))
    xl, xr, p1 = _tc_layer(accn, den8, r(bo0), Wl1, r(bl1), Wr1, r(br1), batch2d)

    accn, den8 = split_acc(_edge_phase(xl, xr, src, dst, ea, We1, att1))
    xl, xr, p2 = _tc_layer(accn, den8, r(bo1), Wl2, r(bl2), Wr2, r(br2), batch2d)

    accn, den8 = split_acc(_edge_phase(xl, xr, src, dst, ea, We2, att2))
    return _tc_final(accn, den8, r(bo2), batch2d, p1, p2, cnt)


# speedup vs baseline: 4.7544x; 1.0428x over previous
import functools

import jax
import jax.numpy as jnp
from jax import lax
from jax.experimental import pallas as pl
from jax.experimental.pallas import tpu as pltpu
from jax.experimental.pallas import tpu_sc as plsc

N = 10000
E = 320000
D = 128
H = 8
C = 16
G = 16
NEG_SLOPE = 0.2
BLK = 2000
GRID = N // BLK

DEN0 = N
NROW = N // 16
N2 = 10752


def _onehot(batch_blk):
    iota = lax.broadcasted_iota(jnp.int32, (BLK, G), 1)
    return (batch_blk == iota).astype(jnp.float32)


def _tc0_body(x_ref, w1_ref, b1_ref, w2_ref, b2_ref, wl_ref, bl_ref,
              wr_ref, br_ref, batch_ref, xl_ref, xr_ref, cnt_ref):
    i = pl.program_id(0)
    x = x_ref[...]
    h = jnp.maximum(jnp.dot(x, w1_ref[...], preferred_element_type=jnp.float32)
                    + b1_ref[...], 0.0)
    h = jnp.maximum(jnp.dot(h, w2_ref[...], preferred_element_type=jnp.float32)
                    + b2_ref[...], 0.0)
    xl_ref[...] = jnp.dot(h, wl_ref[...], preferred_element_type=jnp.float32) + bl_ref[...]
    xr_ref[...] = jnp.dot(h, wr_ref[...], preferred_element_type=jnp.float32) + br_ref[...]
    oh = _onehot(batch_ref[...])
    contrib = lax.dot_general(oh, jnp.ones((BLK, D), jnp.float32),
                              (((0,), (0,)), ((), ())),
                              preferred_element_type=jnp.float32)

    @pl.when(i == 0)
    def _():
        cnt_ref[...] = jnp.zeros_like(cnt_ref)

    cnt_ref[...] += contrib


def _tc0(x, w1, b1, w2, b2, wl, bl, wr, br, batch2d):
    full = lambda s: pl.BlockSpec(s, lambda i: tuple(0 for _ in s))
    return pl.pallas_call(
        _tc0_body,
        grid=(GRID,),
        in_specs=[
            pl.BlockSpec((BLK, D), lambda i: (i, 0)),
            full((D, D)), full((1, D)), full((D, D)), full((1, D)),
            full((D, D)), full((1, D)), full((D, D)), full((1, D)),
            pl.BlockSpec((BLK, 1), lambda i: (i, 0)),
        ],
        out_specs=[
            pl.BlockSpec((BLK, D), lambda i: (i, 0)),
            pl.BlockSpec((BLK, D), lambda i: (i, 0)),
            pl.BlockSpec((G, D), lambda i: (0, 0)),
        ],
        out_shape=[
            jax.ShapeDtypeStruct((N, D), jnp.float32),
            jax.ShapeDtypeStruct((N, D), jnp.float32),
            jax.ShapeDtypeStruct((G, D), jnp.float32),
        ],
    )(x, w1, b1, w2, b2, wl, bl, wr, br, batch2d)


def _norm_h(accn, den8, bo):
    numer = accn[0] + accn[1]
    den = den8[0] + den8[1]
    hh = lax.broadcasted_iota(jnp.int32, (H, D), 0)
    cc = lax.broadcasted_iota(jnp.int32, (H, D), 1)
    expand = (hh == cc // C).astype(jnp.float32)
    den_full = jnp.dot(den, expand, preferred_element_type=jnp.float32)
    return jnp.maximum(numer / (den_full + 1e-16) + bo, 0.0)


def _tc_layer_body(accn_ref, den_ref, bo_ref, wl_ref, bl_ref, wr_ref, br_ref,
                   batch_ref, xl_ref, xr_ref, pool_ref):
    i = pl.program_id(0)
    h = _norm_h(accn_ref[...], den_ref[...], bo_ref[...])
    xl_ref[...] = jnp.dot(h, wl_ref[...], preferred_element_type=jnp.float32) + bl_ref[...]
    xr_ref[...] = jnp.dot(h, wr_ref[...], preferred_element_type=jnp.float32) + br_ref[...]
    oh = _onehot(batch_ref[...])
    contrib = lax.dot_general(oh, h, (((0,), (0,)), ((), ())),
                              preferred_element_type=jnp.float32)

    @pl.when(i == 0)
    def _():
        pool_ref[...] = jnp.zeros_like(pool_ref)

    pool_ref[...] += contrib


def _tc_layer(accn, den8, bo, wl, bl, wr, br, batch2d):
    full = lambda s: pl.BlockSpec(s, lambda i: tuple(0 for _ in s))
    return pl.pallas_call(
        _tc_layer_body,
        grid=(GRID,),
        in_specs=[
            pl.BlockSpec((2, BLK, D), lambda i: (0, i, 0)),
            pl.BlockSpec((2, BLK, H), lambda i: (0, i, 0)),
            full((1, D)),
            full((D, D)), full((1, D)), full((D, D)), full((1, D)),
            pl.BlockSpec((BLK, 1), lambda i: (i, 0)),
        ],
        out_specs=[
            pl.BlockSpec((BLK, D), lambda i: (i, 0)),
            pl.BlockSpec((BLK, D), lambda i: (i, 0)),
            pl.BlockSpec((G, D), lambda i: (0, 0)),
        ],
        out_shape=[
            jax.ShapeDtypeStruct((N, D), jnp.float32),
            jax.ShapeDtypeStruct((N, D), jnp.float32),
            jax.ShapeDtypeStruct((G, D), jnp.float32),
        ],
    )(accn, den8, bo, wl, bl, wr, br, batch2d)


def _tc_final_body(accn_ref, den_ref, bo_ref, batch_ref, p1_ref, p2_ref,
                   cnt_ref, out_ref, pool_ref):
    i = pl.program_id(0)
    h = _norm_h(accn_ref[...], den_ref[...], bo_ref[...])
    oh = _onehot(batch_ref[...])
    contrib = lax.dot_general(oh, h, (((0,), (0,)), ((), ())),
                              preferred_element_type=jnp.float32)

    @pl.when(i == 0)
    def _():
        pool_ref[...] = jnp.zeros_like(pool_ref)

    pool_ref[...] += contrib

    @pl.when(i == GRID - 1)
    def _():
        cnt = jnp.maximum(cnt_ref[...], 1.0)
        out_ref[...] = jnp.concatenate(
            [p1_ref[...] / cnt, p2_ref[...] / cnt, pool_ref[...] / cnt], axis=1)


def _tc_final(accn, den8, bo, batch2d, p1, p2, cnt):
    full = lambda s: pl.BlockSpec(s, lambda i: tuple(0 for _ in s))
    return pl.pallas_call(
        _tc_final_body,
        grid=(GRID,),
        in_specs=[
            pl.BlockSpec((2, BLK, D), lambda i: (0, i, 0)),
            pl.BlockSpec((2, BLK, H), lambda i: (0, i, 0)),
            full((1, D)),
            pl.BlockSpec((BLK, 1), lambda i: (i, 0)),
            full((G, D)), full((G, D)), full((G, D)),
        ],
        out_specs=[
            pl.BlockSpec((G, 3 * D), lambda i: (0, 0)),
            pl.BlockSpec((G, D), lambda i: (0, 0)),
        ],
        out_shape=[
            jax.ShapeDtypeStruct((G, 3 * D), jnp.float32),
            jax.ShapeDtypeStruct((G, D), jnp.float32),
        ],
    )(accn, den8, bo, batch2d, p1, p2, cnt)[0]


NC = 2
NS = 16
NT = NC * NS
EPT = E // NT
K = 80
NG = K // 16
SB = 400
CPS = SB // K
NSUPER = EPT // SB
TPT = N2 // NS
RB = 56


def _sc_edge_body(xl_h, xr_h, src_h, dst_h, ea_h, we_h, att_h, out_h,
                  srcb, dstb, eab, idx2, xs, xd, contrib2,
                  we_v, att_v, shared, sem1, sem2, sem3):
    cid = lax.axis_index("c")
    sid = lax.axis_index("s")
    wid = cid * NS + sid
    zeros16 = jnp.zeros((16,), jnp.float32)
    iota16 = lax.broadcasted_iota(jnp.int32, (16,), 0)
    rows = [iota16 + g * 16 for g in range(NG)]
    rows2 = [r * 2 for r in rows]

    @plsc.parallel_loop(0, 2 * K, unroll=8)
    def _(i):
        for j in range(D // 16):
            contrib2[i, pl.ds(j * 16, 16)] = zeros16

    zdescs = []
    zoff = 0
    for zrows in (2 * K, 2 * K, 2 * K, 2 * K, TPT - 8 * K):
        zdescs.append(pltpu.async_copy(
            contrib2.at[pl.ds(0, zrows)],
            shared.at[pl.ds(sid * TPT + zoff, zrows)], sem1))
        zoff += zrows
    for dsc in zdescs:
        dsc.wait()

    pltpu.sync_copy(we_h, we_v)
    pltpu.sync_copy(att_h, att_v)
    plsc.subcore_barrier()

    ebase = wid * EPT

    @pl.loop(0, NSUPER)
    def _(sc):
        sbase = ebase + sc * SB
        pltpu.sync_copy(src_h.at[pl.ds(sbase, SB)], srcb)
        pltpu.sync_copy(dst_h.at[pl.ds(sbase, SB)], dstb)
        pltpu.sync_copy(ea_h.at[pl.ds(sbase, SB)], eab)

        @pl.loop(0, CPS)
        def _(cc):
            co = cc * K
            d1 = pltpu.async_copy(xl_h.at[srcb.at[pl.ds(co, K)]], xs, sem1)
            d2 = pltpu.async_copy(xr_h.at[dstb.at[pl.ds(co, K)]], xd, sem2)

            @pl.when(jnp.logical_or(sc > 0, cc > 0))
            def _():
                pltpu.make_async_copy(contrib2, shared.at[idx2], sem3).wait()

            d1.wait()
            d2.wait()

            @plsc.parallel_loop(0, K, unroll=8)
            def _(i):
                for j in range(D // 16):
                    contrib2[2 * i + 1, pl.ds(j * 16, 16)] = zeros16

            a_vecs = [eab[pl.ds(co + g * 16, 16)] for g in range(NG)]
            dvs = [dstb[pl.ds(co + g * 16, 16)] for g in range(NG)]
            dencols = [lax.shift_left(lax.bitwise_and(dv, 15), 3) for dv in dvs]
            for g in range(NG):
                plsc.store_scatter(idx2, [rows2[g]], dvs[g])
                plsc.store_scatter(idx2, [rows2[g] + 1],
                                   DEN0 + lax.shift_right_logical(dvs[g], 4))

            for h in range(H):
                col0 = h * C

                accs0 = tuple(jnp.zeros((16,), jnp.float32)
                              for _ in range(NG))

                @plsc.parallel_loop(0, C, unroll=8, carry=accs0)
                def alpha_loop(c, accs, _col0=col0):
                    colv = lax.bitwise_and(iota16 + c, C - 1) + _col0
                    web = plsc.load_gather(we_v, [colv])
                    atb = plsc.load_gather(att_v, [colv])
                    out = []
                    for g in range(NG):
                        xsc = plsc.load_gather(xs, [rows[g], colv])
                        xdc = plsc.load_gather(xd, [rows[g], colv])
                        e = xsc + xdc + a_vecs[g] * web
                        el = jnp.maximum(e, NEG_SLOPE * e)
                        out.append(accs[g] + el * atb)
                    return tuple(out)

                accs = alpha_loop
                exs = [jnp.exp(a) for a in accs]
                for g in range(NG):
                    plsc.store_scatter(contrib2, [rows2[g] + 1, dencols[g] + h],
                                       exs[g])

                @plsc.parallel_loop(0, C, unroll=8)
                def _(c, _col0=col0, _exs=exs):
                    colv = lax.bitwise_and(iota16 + c, C - 1) + _col0
                    for g in range(NG):
                        xsc = plsc.load_gather(xs, [rows[g], colv])
                        plsc.store_scatter(contrib2, [rows2[g], colv],
                                           xsc * _exs[g])

            pltpu.async_copy(contrib2, shared.at[idx2], sem3, add=True)

    pltpu.make_async_copy(contrib2, shared.at[idx2], sem3).wait()
    plsc.subcore_barrier()

    r0 = sid * TPT
    pltpu.sync_copy(shared.at[pl.ds(r0, TPT)], out_h.at[cid, pl.ds(r0, TPT)])


def _edge_phase(xl, xr, src, dst, ea, we_flat, att_flat):
    mesh = plsc.VectorSubcoreMesh(core_axis_name="c", subcore_axis_name="s")
    f = pl.kernel(
        _sc_edge_body,
        out_type=jax.ShapeDtypeStruct((NC, N2, D), jnp.float32),
        mesh=mesh,
        compiler_params=pltpu.CompilerParams(needs_layout_passes=False),
        scratch_types=[
            pltpu.VMEM((SB,), jnp.int32),
            pltpu.VMEM((SB,), jnp.int32),
            pltpu.VMEM((SB,), jnp.float32),
            pltpu.VMEM((2 * K,), jnp.int32),
            pltpu.VMEM((K, D), jnp.float32),
            pltpu.VMEM((K, D), jnp.float32),
            pltpu.VMEM((2 * K, D), jnp.float32),
            pltpu.VMEM((D,), jnp.float32),
            pltpu.VMEM((D,), jnp.float32),
            pltpu.VMEM_SHARED((N2, D), jnp.float32),
            pltpu.SemaphoreType.DMA,
            pltpu.SemaphoreType.DMA,
            pltpu.SemaphoreType.DMA,
        ],
    )
    return f(xl, xr, src, dst, ea, we_flat.reshape(D), att_flat.reshape(D))


def kernel(x, edge_index, edge_attr, batch, W_pre1, b_pre1, W_pre2, b_pre2,
           Wl0, bl0, Wr0, br0, We0, att0, bo0,
           Wl1, bl1, Wr1, br1, We1, att1, bo1,
           Wl2, bl2, Wr2, br2, We2, att2, bo2):
    src = edge_index[0]
    dst = edge_index[1]
    ea = edge_attr.reshape(E)
    batch2d = batch.reshape(N, 1)
    r = lambda b: b.reshape(1, D)

    def split_acc(acc):
        accn = acc[:, :N, :]
        den8 = acc[:, DEN0:DEN0 + NROW, :].reshape(2, NROW, 16, H).reshape(2, N, H)
        return accn, den8

    xl, xr, cnt = _tc0(x, W_pre1, r(b_pre1), W_pre2, r(b_pre2),
                       Wl0, r(bl0), Wr0, r(br0), batch2d)

    accn, den8 = split_acc(_edge_phase(xl, xr, src, dst, ea, We0, att0))
    xl, xr, p1 = _tc_layer(accn, den8, r(bo0), Wl1, r(bl1), Wr1, r(br1), batch2d)

    accn, den8 = split_acc(_edge_phase(xl, xr, src, dst, ea, We1, att1))
    xl, xr, p2 = _tc_layer(accn, den8, r(bo1), Wl2, r(bl2), Wr2, r(br2), batch2d)

    accn, den8 = split_acc(_edge_phase(xl, xr, src, dst, ea, We2, att2))
    return _tc_final(accn, den8, r(bo2), batch2d, p1, p2, cnt)


# speedup vs baseline: 4.9607x; 1.0434x over previous
import functools

import jax
import jax.numpy as jnp
from jax import lax
from jax.experimental import pallas as pl
from jax.experimental.pallas import tpu as pltpu
from jax.experimental.pallas import tpu_sc as plsc

N = 10000
E = 320000
D = 128
H = 8
C = 16
G = 16
NEG_SLOPE = 0.2
BLK = 2000
GRID = N // BLK

DEN0 = N
NROW = N // 16
N2 = 10752


def _onehot(batch_blk):
    iota = lax.broadcasted_iota(jnp.int32, (BLK, G), 1)
    return (batch_blk == iota).astype(jnp.float32)


def _tc0_body(x_ref, w1_ref, b1_ref, w2_ref, b2_ref, wl_ref, bl_ref,
              wr_ref, br_ref, batch_ref, xl_ref, xr_ref, cnt_ref):
    i = pl.program_id(0)
    x = x_ref[...]
    h = jnp.maximum(jnp.dot(x, w1_ref[...], preferred_element_type=jnp.float32)
                    + b1_ref[...], 0.0)
    h = jnp.maximum(jnp.dot(h, w2_ref[...], preferred_element_type=jnp.float32)
                    + b2_ref[...], 0.0)
    xl_ref[...] = jnp.dot(h, wl_ref[...], preferred_element_type=jnp.float32) + bl_ref[...]
    xr_ref[...] = jnp.dot(h, wr_ref[...], preferred_element_type=jnp.float32) + br_ref[...]
    oh = _onehot(batch_ref[...])
    contrib = lax.dot_general(oh, jnp.ones((BLK, D), jnp.float32),
                              (((0,), (0,)), ((), ())),
                              preferred_element_type=jnp.float32)

    @pl.when(i == 0)
    def _():
        cnt_ref[...] = jnp.zeros_like(cnt_ref)

    cnt_ref[...] += contrib


def _tc0(x, w1, b1, w2, b2, wl, bl, wr, br, batch2d):
    full = lambda s: pl.BlockSpec(s, lambda i: tuple(0 for _ in s))
    return pl.pallas_call(
        _tc0_body,
        grid=(GRID,),
        in_specs=[
            pl.BlockSpec((BLK, D), lambda i: (i, 0)),
            full((D, D)), full((1, D)), full((D, D)), full((1, D)),
            full((D, D)), full((1, D)), full((D, D)), full((1, D)),
            pl.BlockSpec((BLK, 1), lambda i: (i, 0)),
        ],
        out_specs=[
            pl.BlockSpec((BLK, D), lambda i: (i, 0)),
            pl.BlockSpec((BLK, D), lambda i: (i, 0)),
            pl.BlockSpec((G, D), lambda i: (0, 0)),
        ],
        out_shape=[
            jax.ShapeDtypeStruct((N, D), jnp.float32),
            jax.ShapeDtypeStruct((N, D), jnp.float32),
            jax.ShapeDtypeStruct((G, D), jnp.float32),
        ],
    )(x, w1, b1, w2, b2, wl, bl, wr, br, batch2d)


def _norm_h(accn, den8, bo):
    numer = accn[0] + accn[1]
    den = den8[0] + den8[1]
    hh = lax.broadcasted_iota(jnp.int32, (H, D), 0)
    cc = lax.broadcasted_iota(jnp.int32, (H, D), 1)
    expand = (hh == cc // C).astype(jnp.float32)
    den_full = jnp.dot(den, expand, preferred_element_type=jnp.float32)
    return jnp.maximum(numer / (den_full + 1e-16) + bo, 0.0)


def _tc_layer_body(accn_ref, den_ref, bo_ref, wl_ref, bl_ref, wr_ref, br_ref,
                   batch_ref, xl_ref, xr_ref, pool_ref):
    i = pl.program_id(0)
    h = _norm_h(accn_ref[...], den_ref[...], bo_ref[...])
    xl_ref[...] = jnp.dot(h, wl_ref[...], preferred_element_type=jnp.float32) + bl_ref[...]
    xr_ref[...] = jnp.dot(h, wr_ref[...], preferred_element_type=jnp.float32) + br_ref[...]
    oh = _onehot(batch_ref[...])
    contrib = lax.dot_general(oh, h, (((0,), (0,)), ((), ())),
                              preferred_element_type=jnp.float32)

    @pl.when(i == 0)
    def _():
        pool_ref[...] = jnp.zeros_like(pool_ref)

    pool_ref[...] += contrib


def _tc_layer(accn, den8, bo, wl, bl, wr, br, batch2d):
    full = lambda s: pl.BlockSpec(s, lambda i: tuple(0 for _ in s))
    return pl.pallas_call(
        _tc_layer_body,
        grid=(GRID,),
        in_specs=[
            pl.BlockSpec((2, BLK, D), lambda i: (0, i, 0)),
            pl.BlockSpec((2, BLK, H), lambda i: (0, i, 0)),
            full((1, D)),
            full((D, D)), full((1, D)), full((D, D)), full((1, D)),
            pl.BlockSpec((BLK, 1), lambda i: (i, 0)),
        ],
        out_specs=[
            pl.BlockSpec((BLK, D), lambda i: (i, 0)),
            pl.BlockSpec((BLK, D), lambda i: (i, 0)),
            pl.BlockSpec((G, D), lambda i: (0, 0)),
        ],
        out_shape=[
            jax.ShapeDtypeStruct((N, D), jnp.float32),
            jax.ShapeDtypeStruct((N, D), jnp.float32),
            jax.ShapeDtypeStruct((G, D), jnp.float32),
        ],
    )(accn, den8, bo, wl, bl, wr, br, batch2d)


def _tc_final_body(accn_ref, den_ref, bo_ref, batch_ref, p1_ref, p2_ref,
                   cnt_ref, out_ref, pool_ref):
    i = pl.program_id(0)
    h = _norm_h(accn_ref[...], den_ref[...], bo_ref[...])
    oh = _onehot(batch_ref[...])
    contrib = lax.dot_general(oh, h, (((0,), (0,)), ((), ())),
                              preferred_element_type=jnp.float32)

    @pl.when(i == 0)
    def _():
        pool_ref[...] = jnp.zeros_like(pool_ref)

    pool_ref[...] += contrib

    @pl.when(i == GRID - 1)
    def _():
        cnt = jnp.maximum(cnt_ref[...], 1.0)
        out_ref[...] = jnp.concatenate(
            [p1_ref[...] / cnt, p2_ref[...] / cnt, pool_ref[...] / cnt], axis=1)


def _tc_final(accn, den8, bo, batch2d, p1, p2, cnt):
    full = lambda s: pl.BlockSpec(s, lambda i: tuple(0 for _ in s))
    return pl.pallas_call(
        _tc_final_body,
        grid=(GRID,),
        in_specs=[
            pl.BlockSpec((2, BLK, D), lambda i: (0, i, 0)),
            pl.BlockSpec((2, BLK, H), lambda i: (0, i, 0)),
            full((1, D)),
            pl.BlockSpec((BLK, 1), lambda i: (i, 0)),
            full((G, D)), full((G, D)), full((G, D)),
        ],
        out_specs=[
            pl.BlockSpec((G, 3 * D), lambda i: (0, 0)),
            pl.BlockSpec((G, D), lambda i: (0, 0)),
        ],
        out_shape=[
            jax.ShapeDtypeStruct((G, 3 * D), jnp.float32),
            jax.ShapeDtypeStruct((G, D), jnp.float32),
        ],
    )(accn, den8, bo, batch2d, p1, p2, cnt)[0]


NC = 2
NS = 16
NT = NC * NS
EPT = E // NT
K = 80
NG = K // 16
SB = 400
CPS = SB // K
NSUPER = EPT // SB
TPT = N2 // NS
RB = 56


def _sc_edge_body(xl_h, xr_h, src_h, dst_h, ea_h, we_h, att_h, out_h,
                  srcb, dstb, eab, idx2, xs, xd, contrib2,
                  we_v, att_v, shared, sem1, sem2, sem3):
    cid = lax.axis_index("c")
    sid = lax.axis_index("s")
    wid = cid * NS + sid
    zeros16 = jnp.zeros((16,), jnp.float32)
    iota16 = lax.broadcasted_iota(jnp.int32, (16,), 0)
    rows = [iota16 + g * 16 for g in range(NG)]
    rows2 = [r * 2 for r in rows]

    @plsc.parallel_loop(0, 2 * K, unroll=8)
    def _(i):
        for j in range(D // 16):
            contrib2[i, pl.ds(j * 16, 16)] = zeros16

    zdescs = []
    zoff = 0
    for zrows in (2 * K, 2 * K, 2 * K, 2 * K, TPT - 8 * K):
        zdescs.append(pltpu.async_copy(
            contrib2.at[pl.ds(0, zrows)],
            shared.at[pl.ds(sid * TPT + zoff, zrows)], sem1))
        zoff += zrows
    for dsc in zdescs:
        dsc.wait()

    pltpu.sync_copy(we_h, we_v)
    pltpu.sync_copy(att_h, att_v)
    plsc.subcore_barrier()

    ebase = wid * EPT

    @pl.loop(0, NSUPER)
    def _(sc):
        sbase = ebase + sc * SB
        pltpu.sync_copy(src_h.at[pl.ds(sbase, SB)], srcb)
        pltpu.sync_copy(dst_h.at[pl.ds(sbase, SB)], dstb)
        pltpu.sync_copy(ea_h.at[pl.ds(sbase, SB)], eab)

        @pl.loop(0, CPS)
        def _(cc):
            co = cc * K
            d1 = pltpu.async_copy(xl_h.at[srcb.at[pl.ds(co, K)]], xs, sem1)
            d2 = pltpu.async_copy(xr_h.at[dstb.at[pl.ds(co, K)]], xd, sem2)

            @pl.when(jnp.logical_or(sc > 0, cc > 0))
            def _():
                pltpu.make_async_copy(contrib2, shared.at[idx2], sem3).wait()

            @plsc.parallel_loop(0, K, unroll=8)
            def _(i):
                for j in range(D // 16):
                    contrib2[2 * i + 1, pl.ds(j * 16, 16)] = zeros16

            a_vecs = [eab[pl.ds(co + g * 16, 16)] for g in range(NG)]
            dvs = [dstb[pl.ds(co + g * 16, 16)] for g in range(NG)]
            dencols = [lax.shift_left(lax.bitwise_and(dv, 15), 3) for dv in dvs]
            for g in range(NG):
                plsc.store_scatter(idx2, [rows2[g]], dvs[g])
                plsc.store_scatter(idx2, [rows2[g] + 1],
                                   DEN0 + lax.shift_right_logical(dvs[g], 4))

            d1.wait()
            d2.wait()

            for h in range(H):
                col0 = h * C

                accs0 = tuple(jnp.zeros((16,), jnp.float32)
                              for _ in range(NG))

                @plsc.parallel_loop(0, C, unroll=8, carry=accs0)
                def alpha_loop(c, accs, _col0=col0):
                    colv = lax.bitwise_and(iota16 + c, C - 1) + _col0
                    web = plsc.load_gather(we_v, [colv])
                    atb = plsc.load_gather(att_v, [colv])
                    out = []
                    for g in range(NG):
                        xsc = plsc.load_gather(xs, [rows[g], colv])
                        xdc = plsc.load_gather(xd, [rows[g], colv])
                        e = xsc + xdc + a_vecs[g] * web
                        el = jnp.maximum(e, NEG_SLOPE * e)
                        out.append(accs[g] + el * atb)
                    return tuple(out)

                accs = alpha_loop
                exs = [jnp.exp(a) for a in accs]
                for g in range(NG):
                    plsc.store_scatter(contrib2, [rows2[g] + 1, dencols[g] + h],
                                       exs[g])

                @plsc.parallel_loop(0, C, unroll=8)
                def _(c, _col0=col0, _exs=exs):
                    colv = lax.bitwise_and(iota16 + c, C - 1) + _col0
                    for g in range(NG):
                        xsc = plsc.load_gather(xs, [rows[g], colv])
                        plsc.store_scatter(contrib2, [rows2[g], colv],
                                           xsc * _exs[g])

            pltpu.async_copy(contrib2, shared.at[idx2], sem3, add=True)

    pltpu.make_async_copy(contrib2, shared.at[idx2], sem3).wait()
    plsc.subcore_barrier()

    r0 = sid * TPT
    pltpu.sync_copy(shared.at[pl.ds(r0, TPT)], out_h.at[cid, pl.ds(r0, TPT)])


def _edge_phase(xl, xr, src, dst, ea, we_flat, att_flat):
    mesh = plsc.VectorSubcoreMesh(core_axis_name="c", subcore_axis_name="s")
    f = pl.kernel(
        _sc_edge_body,
        out_type=jax.ShapeDtypeStruct((NC, N2, D), jnp.float32),
        mesh=mesh,
        compiler_params=pltpu.CompilerParams(needs_layout_passes=False),
        scratch_types=[
            pltpu.VMEM((SB,), jnp.int32),
            pltpu.VMEM((SB,), jnp.int32),
            pltpu.VMEM((SB,), jnp.float32),
            pltpu.VMEM((2 * K,), jnp.int32),
            pltpu.VMEM((K, D), jnp.float32),
            pltpu.VMEM((K, D), jnp.float32),
            pltpu.VMEM((2 * K, D), jnp.float32),
            pltpu.VMEM((D,), jnp.float32),
            pltpu.VMEM((D,), jnp.float32),
            pltpu.VMEM_SHARED((N2, D), jnp.float32),
            pltpu.SemaphoreType.DMA,
            pltpu.SemaphoreType.DMA,
            pltpu.SemaphoreType.DMA,
        ],
    )
    return f(xl, xr, src, dst, ea, we_flat.reshape(D), att_flat.reshape(D))


def kernel(x, edge_index, edge_attr, batch, W_pre1, b_pre1, W_pre2, b_pre2,
           Wl0, bl0, Wr0, br0, We0, att0, bo0,
           Wl1, bl1, Wr1, br1, We1, att1, bo1,
           Wl2, bl2, Wr2, br2, We2, att2, bo2):
    src = edge_index[0]
    dst = edge_index[1]
    ea = edge_attr.reshape(E)
    batch2d = batch.reshape(N, 1)
    r = lambda b: b.reshape(1, D)

    def split_acc(acc):
        accn = acc[:, :N, :]
        den8 = acc[:, DEN0:DEN0 + NROW, :].reshape(2, NROW, 16, H).reshape(2, N, H)
        return accn, den8

    xl, xr, cnt = _tc0(x, W_pre1, r(b_pre1), W_pre2, r(b_pre2),
                       Wl0, r(bl0), Wr0, r(br0), batch2d)

    accn, den8 = split_acc(_edge_phase(xl, xr, src, dst, ea, We0, att0))
    xl, xr, p1 = _tc_layer(accn, den8, r(bo0), Wl1, r(bl1), Wr1, r(br1), batch2d)

    accn, den8 = split_acc(_edge_phase(xl, xr, src, dst, ea, We1, att1))
    xl, xr, p2 = _tc_layer(accn, den8, r(bo1), Wl2, r(bl2), Wr2, r(br2), batch2d)

    accn, den8 = split_acc(_edge_phase(xl, xr, src, dst, ea, We2, att2))
    return _tc_final(accn, den8, r(bo2), batch2d, p1, p2, cnt)
